# Initial kernel scaffold; baseline (speedup 1.0000x reference)
import jax
import jax.numpy as jnp
from jax.experimental import pallas as pl


def kernel(x, gate, w1, w2, w3, sw1, sw2, sw3):
    raise NotImplementedError("write your pallas kernel here")


# speedup vs baseline: 1.1203x; 1.1203x over previous
import functools
import jax
import jax.numpy as jnp
from jax.experimental import pallas as pl
from jax.experimental.pallas import tpu as pltpu

E = 8
TOPK = 2


def _router_body(x_ref, gate_ref, cmb_ref):
    x = x_ref[...]
    logits = jax.lax.dot_general(
        x, gate_ref[...], (((1,), (1,)), ((), ())),
        preferred_element_type=jnp.float32)
    T = logits.shape[0]
    eidx = jax.lax.broadcasted_iota(jnp.int32, (T, E), 1)
    m1 = jnp.max(logits, axis=1, keepdims=True)
    e1 = jnp.min(jnp.where(logits == m1, eidx, E), axis=1, keepdims=True)
    masked = jnp.where(eidx == e1, -jnp.inf, logits)
    m2 = jnp.max(masked, axis=1, keepdims=True)
    e2 = jnp.min(jnp.where(masked == m2, eidx, E), axis=1, keepdims=True)
    z = jnp.sum(jnp.exp(logits - m1), axis=1, keepdims=True)
    wa = 1.0 / z
    wb = jnp.exp(m2 - m1) / z
    eidx9 = jax.lax.broadcasted_iota(jnp.int32, (T, E + 1), 1)
    cmb = (jnp.where(eidx9 == e1, wa, 0.0)
           + jnp.where(eidx9 == e2, wb, 0.0)
           + jnp.where(eidx9 == E, 1.0, 0.0))
    cmb_ref[...] = cmb


def _moe_body(cmb_ref, x_ref, w1_ref, w3_ref, w2_ref, out_ref):
    e = pl.program_id(0)
    x = x_ref[...]
    cmb = cmb_ref[...]
    eidx = jax.lax.broadcasted_iota(jnp.int32, cmb.shape, 1)
    c = jnp.sum(jnp.where(eidx == e, cmb, 0.0), axis=1, keepdims=True)
    h = jax.lax.dot_general(x, w1_ref[0], (((1,), (1,)), ((), ())),
                            preferred_element_type=jnp.float32)
    g = jax.lax.dot_general(x, w3_ref[0], (((1,), (1,)), ((), ())),
                            preferred_element_type=jnp.float32)
    a = h * jax.lax.logistic(h) * g
    y = jax.lax.dot_general(a, w2_ref[0], (((1,), (1,)), ((), ())),
                            preferred_element_type=jnp.float32)
    contrib = c * y

    @pl.when(e == 0)
    def _():
        out_ref[...] = contrib

    @pl.when(e != 0)
    def _():
        out_ref[...] += contrib


def kernel(x, gate, w1, w2, w3, sw1, sw2, sw3):
    bs, slen, dim = x.shape
    xt = x.reshape(-1, dim)
    T = xt.shape[0]
    hidden = w1.shape[1]

    cmb = pl.pallas_call(
        _router_body,
        out_shape=jax.ShapeDtypeStruct((T, E + 1), jnp.float32),
    )(xt, gate)

    w1c = jnp.concatenate([w1, sw1[None]], axis=0)
    w3c = jnp.concatenate([w3, sw3[None]], axis=0)
    w2c = jnp.concatenate([w2, sw2[None]], axis=0)

    out = pl.pallas_call(
        _moe_body,
        grid=(E + 1,),
        in_specs=[
            pl.BlockSpec((T, E + 1), lambda e: (0, 0)),
            pl.BlockSpec((T, dim), lambda e: (0, 0)),
            pl.BlockSpec((1, hidden, dim), lambda e: (e, 0, 0)),
            pl.BlockSpec((1, hidden, dim), lambda e: (e, 0, 0)),
            pl.BlockSpec((1, dim, hidden), lambda e: (e, 0, 0)),
        ],
        out_specs=pl.BlockSpec((T, dim), lambda e: (0, 0)),
        out_shape=jax.ShapeDtypeStruct((T, dim), jnp.float32),
    )(cmb, xt, w1c, w3c, w2c)

    return out.reshape(bs, slen, dim)


# speedup vs baseline: 1.1205x; 1.0002x over previous
import functools
import jax
import jax.numpy as jnp
from jax.experimental import pallas as pl
from jax.experimental.pallas import tpu as pltpu

E = 8
TOPK = 2


def _router_body(x_ref, gate_ref, cmb_ref):
    x = x_ref[...]
    logits = jax.lax.dot_general(
        x, gate_ref[...], (((1,), (1,)), ((), ())),
        preferred_element_type=jnp.float32)
    T = logits.shape[0]
    eidx = jax.lax.broadcasted_iota(jnp.int32, (T, E), 1)
    m1 = jnp.max(logits, axis=1, keepdims=True)
    e1 = jnp.min(jnp.where(logits == m1, eidx, E), axis=1, keepdims=True)
    masked = jnp.where(eidx == e1, -jnp.inf, logits)
    m2 = jnp.max(masked, axis=1, keepdims=True)
    e2 = jnp.min(jnp.where(masked == m2, eidx, E), axis=1, keepdims=True)
    z = jnp.sum(jnp.exp(logits - m1), axis=1, keepdims=True)
    wa = 1.0 / z
    wb = jnp.exp(m2 - m1) / z
    eidx9 = jax.lax.broadcasted_iota(jnp.int32, (T, E + 1), 1)
    cmb = (jnp.where(eidx9 == e1, wa, 0.0)
           + jnp.where(eidx9 == e2, wb, 0.0)
           + jnp.where(eidx9 == E, 1.0, 0.0))
    cmb_ref[...] = cmb


def _moe_body(cmb_ref, x_ref, w1_ref, w3_ref, w2_ref, out_ref):
    e = pl.program_id(1)
    x = x_ref[...].astype(jnp.bfloat16)
    cmb = cmb_ref[...]
    eidx = jax.lax.broadcasted_iota(jnp.int32, cmb.shape, 1)
    c = jnp.sum(jnp.where(eidx == e, cmb, 0.0), axis=1, keepdims=True)
    h = jax.lax.dot_general(x, w1_ref[0], (((1,), (1,)), ((), ())),
                            preferred_element_type=jnp.float32)
    g = jax.lax.dot_general(x, w3_ref[0], (((1,), (1,)), ((), ())),
                            preferred_element_type=jnp.float32)
    a = (h * jax.lax.logistic(h) * g).astype(jnp.bfloat16)
    y = jax.lax.dot_general(a, w2_ref[0], (((1,), (1,)), ((), ())),
                            preferred_element_type=jnp.float32)
    contrib = c * y

    @pl.when(e == 0)
    def _():
        out_ref[...] = contrib

    @pl.when(e != 0)
    def _():
        out_ref[...] += contrib


def kernel(x, gate, w1, w2, w3, sw1, sw2, sw3):
    bs, slen, dim = x.shape
    xt = x.reshape(-1, dim)
    T = xt.shape[0]
    hidden = w1.shape[1]

    cmb = pl.pallas_call(
        _router_body,
        out_shape=jax.ShapeDtypeStruct((T, E + 1), jnp.float32),
    )(xt, gate)

    bf = jnp.bfloat16
    w1c = jnp.concatenate([w1, sw1[None]], axis=0).astype(bf)
    w3c = jnp.concatenate([w3, sw3[None]], axis=0).astype(bf)
    w2c = jnp.concatenate([w2, sw2[None]], axis=0).astype(bf)

    TB = 1024
    out = pl.pallas_call(
        _moe_body,
        grid=(T // TB, E + 1),
        in_specs=[
            pl.BlockSpec((TB, E + 1), lambda t, e: (t, 0)),
            pl.BlockSpec((TB, dim), lambda t, e: (t, 0)),
            pl.BlockSpec((1, hidden, dim), lambda t, e: (e, 0, 0)),
            pl.BlockSpec((1, hidden, dim), lambda t, e: (e, 0, 0)),
            pl.BlockSpec((1, dim, hidden), lambda t, e: (e, 0, 0)),
        ],
        out_specs=pl.BlockSpec((TB, dim), lambda t, e: (t, 0)),
        out_shape=jax.ShapeDtypeStruct((T, dim), jnp.float32),
    )(cmb, xt, w1c, w3c, w2c)

    return out.reshape(bs, slen, dim)
